# baseline (speedup 1.0000x reference)
import functools

import jax
import jax.numpy as jnp
from jax import lax
from jax.experimental import pallas as pl
from jax.experimental.pallas import tpu as pltpu
from jax.experimental.pallas import tpu_sc as plsc

NN = 50000
NE = 800000
NG = 512
HID = 64
HH = HID // 2
IN_PAD = 8

NCORE = 2
NSUB = 16
L = 16

NP = 50176
TRN = NP // NSUB
ACC_ROWS = NP + 8

EP = 819200
B = 128
ER = EP // B
CPB = 40
EITERS = ER // NSUB
NBLK = EITERS // CPB
DITERS = ER // NSUB // 2
DBLK = DITERS // CPB
NBUF = 4

CH = 112
NCH = TRN // CH

PG = 640
PTT = PG // NSUB

_MESH = plsc.VectorSubcoreMesh(
    core_axis_name="c", subcore_axis_name="s",
    num_cores=NCORE, num_subcores=NSUB)
_SC_PARAMS = pltpu.CompilerParams(use_tc_tiling_on_sc=False)


@functools.partial(
    pl.kernel,
    out_type=jax.ShapeDtypeStruct((NCORE, NP, L), jnp.float32),
    mesh=_MESH,
    compiler_params=_SC_PARAMS,
    scratch_types=[
        pltpu.VMEM((CPB, B), jnp.int32),
        pltpu.VMEM((B, L), jnp.float32),
        pltpu.VMEM((CH, L), jnp.float32),
        pltpu.VMEM_SHARED((ACC_ROWS, L), jnp.float32),
    ],
)
def _deg_kernel(dst_hbm, deg_hbm, dblk, ones, ibuf, cnt):
    c = lax.axis_index("c")
    s = lax.axis_index("s")

    def fill1(r, _):
        ones[r, :] = jnp.full((L,), 1.0, jnp.float32)
        return 0
    lax.fori_loop(0, B, fill1, 0)

    init = jnp.where(c == 0, 1.0, 0.0)

    def filli(r, _):
        ibuf[r, :] = jnp.full((L,), 1.0, jnp.float32) * init
        return 0
    lax.fori_loop(0, CH, filli, 0)

    def initl(k, _):
        pltpu.sync_copy(ibuf, cnt.at[pl.ds(s * TRN + k * CH, CH)])
        return 0
    lax.fori_loop(0, NCH, initl, 0)
    plsc.subcore_barrier()

    def blkloop(blk, _):
        row0 = c * (ER // 2) + s * DITERS + blk * CPB
        pltpu.sync_copy(dst_hbm.at[pl.ds(row0, CPB)], dblk)

        def cloop(jj, _):
            pltpu.sync_copy(ones, cnt.at[dblk.at[jj]], add=True)
            return 0
        lax.fori_loop(0, CPB, cloop, 0)
        return 0
    lax.fori_loop(0, DBLK, blkloop, 0)
    plsc.subcore_barrier()

    pltpu.sync_copy(cnt.at[pl.ds(s * TRN, TRN)],
                    deg_hbm.at[c, pl.ds(s * TRN, TRN)])


def _edge_ring(c, s, hp_hbm, src_hbm, dst_hbm, sblk, dblk, rows, gsem, tsem,
               acc):
    def blkloop(blk, _):
        row0 = s * EITERS + blk * CPB
        pltpu.sync_copy(src_hbm.at[pl.ds(row0, CPB)], sblk)
        pltpu.sync_copy(dst_hbm.at[pl.ds(row0, CPB)], dblk)
        pltpu.async_copy(hp_hbm.at[c].at[sblk.at[0]], rows[0], gsem[0])
        pltpu.async_copy(hp_hbm.at[c].at[sblk.at[1]], rows[1], gsem[1])

        def qloop(t, _):
            for u in range(NBUF):
                j = NBUF * t + u
                v = (u + 2) % NBUF

                @pl.when(j >= 2)
                def _():
                    pltpu.make_async_copy(
                        rows[v], acc.at[pl.ds(0, B)], tsem[v]).wait()

                @pl.when(j + 2 < CPB)
                def _():
                    pltpu.async_copy(
                        hp_hbm.at[c].at[sblk.at[j + 2]], rows[v], gsem[v])

                pltpu.make_async_copy(
                    hp_hbm.at[c, pl.ds(0, B)], rows[u], gsem[u]).wait()
                pltpu.async_copy(
                    rows[u], acc.at[dblk.at[j]], tsem[u], add=True)
            return 0
        lax.fori_loop(0, CPB // NBUF, qloop, 0)
        pltpu.make_async_copy(rows[2], acc.at[pl.ds(0, B)], tsem[2]).wait()
        pltpu.make_async_copy(rows[3], acc.at[pl.ds(0, B)], tsem[3]).wait()
        return 0
    lax.fori_loop(0, NBLK, blkloop, 0)


def _edge_ring_big(c, s, hp_hbm, src_hbm, dst_hbm, sblk, dblk, rows, gsem,
                   tsem, acc):
    def gth(j):
        pltpu.async_copy(hp_hbm.at[c].at[sblk.at[j]],
                         rows[j % 4], gsem[j % 4])

    def wgth(j):
        pltpu.make_async_copy(hp_hbm.at[c, pl.ds(0, B)],
                              rows[j % 4], gsem[j % 4]).wait()

    def sct(j):
        pltpu.async_copy(rows[j % 4], acc.at[dblk.at[j]],
                         tsem[j % 4], add=True)

    def wsct(j):
        pltpu.make_async_copy(rows[j % 4], acc.at[pl.ds(0, B)],
                              tsem[j % 4]).wait()

    def blkloop(blk, _):
        row0 = s * EITERS + blk * CPB
        pltpu.sync_copy(src_hbm.at[pl.ds(row0, CPB)], sblk)
        pltpu.sync_copy(dst_hbm.at[pl.ds(row0, CPB)], dblk)
        gth(0)
        gth(1)
        for j in range(CPB):
            wgth(j)
            sct(j)
            if j + 2 < CPB:
                if j >= 2:
                    wsct(j - 2)
                gth(j + 2)
        for j in range(CPB - 4, CPB):
            wsct(j)
        return 0
    lax.fori_loop(0, NBLK, blkloop, 0)


def _conv_g_body(hp_hbm, src_hbm, dst_hbm, g_hbm,
                 sblk, dblk, r0, r1, r2, r3, acc,
                 g0, g1, g2, g3, t0, t1, t2, t3):
    c = lax.axis_index("c")
    s = lax.axis_index("s")
    pltpu.sync_copy(hp_hbm.at[c, pl.ds(s * TRN, TRN)],
                    acc.at[pl.ds(s * TRN, TRN)])
    plsc.subcore_barrier()
    _edge_ring_big(c, s, hp_hbm, src_hbm, dst_hbm, sblk, dblk,
                   (r0, r1, r2, r3), (g0, g1, g2, g3), (t0, t1, t2, t3), acc)
    plsc.subcore_barrier()
    pltpu.sync_copy(acc.at[pl.ds(s * TRN, TRN)],
                    g_hbm.at[c, pl.ds(s * TRN, TRN)])


def _conv_pool_body(hp_hbm, src_hbm, dst_hbm, dinv_hbm, batch_hbm,
                    pooled_hbm, cntg_hbm,
                    sblk, dblk, r0, r1, r2, r3, dchunk, bidx,
                    acc, pooled, cntg,
                    g0, g1, g2, g3, t0, t1, t2, t3):
    c = lax.axis_index("c")
    s = lax.axis_index("s")
    pltpu.sync_copy(hp_hbm.at[c, pl.ds(s * TRN, TRN)],
                    acc.at[pl.ds(s * TRN, TRN)])
    plsc.subcore_barrier()
    _edge_ring(c, s, hp_hbm, src_hbm, dst_hbm, sblk, dblk,
               (r0, r1, r2, r3), (g0, g1, g2, g3), (t0, t1, t2, t3), acc)

    def fillz(r, _):
        for j in range(HH // L):
            r3[r, pl.ds(j * L, L)] = jnp.zeros((L,), jnp.float32)
        return 0
    lax.fori_loop(0, PTT, fillz, 0)
    pltpu.sync_copy(r3.at[pl.ds(0, PTT)], pooled.at[pl.ds(s * PTT, PTT)])
    pltpu.sync_copy(r3.at[pl.ds(0, PTT)], cntg.at[pl.ds(s * PTT, PTT)])

    def fillo(r, _):
        for j in range(HH // L):
            r2[r, pl.ds(j * L, L)] = jnp.full((L,), 1.0, jnp.float32)
        return 0
    lax.fori_loop(0, CH, fillo, 0)
    plsc.subcore_barrier()

    def nloop(k, _):
        grow = s * TRN + k * CH
        pltpu.sync_copy(acc.at[pl.ds(grow, CH)], r0.at[pl.ds(0, CH)])
        pltpu.sync_copy(dinv_hbm.at[pl.ds(grow, CH)], dchunk)
        pltpu.sync_copy(batch_hbm.at[pl.ds(grow, CH)], bidx)

        def rblk(t, _):
            dv16 = dchunk[pl.ds(t * L, L)]
            for r16 in range(L):
                row = t * L + r16
                dv = dv16[r16]
                for j in range(HH // L):
                    sl = pl.ds(j * L, L)
                    r0[row, sl] = r0[row, sl] * dv
            return 0
        lax.fori_loop(0, CH // L, rblk, 0)

        pltpu.sync_copy(r0.at[pl.ds(0, CH)], pooled.at[bidx], add=True)

        @pl.when(c == 0)
        def _():
            pltpu.sync_copy(r2.at[pl.ds(0, CH)], cntg.at[bidx], add=True)
        return 0
    lax.fori_loop(0, NCH, nloop, 0)
    plsc.subcore_barrier()

    @pl.when(s == 0)
    def _():
        pltpu.sync_copy(pooled.at[pl.ds(0, NG)], pooled_hbm.at[c])

        @pl.when(c == 0)
        def _():
            pltpu.sync_copy(cntg.at[pl.ds(0, NG)], cntg_hbm)


_RING_SCRATCH = [
    pltpu.VMEM((CPB, B), jnp.int32),
    pltpu.VMEM((CPB, B), jnp.int32),
    pltpu.VMEM((B, HH), jnp.float32),
    pltpu.VMEM((B, HH), jnp.float32),
    pltpu.VMEM((B, HH), jnp.float32),
    pltpu.VMEM((B, HH), jnp.float32),
]
_SEMS = [pltpu.SemaphoreType.DMA] * (2 * NBUF)

_conv_g = pl.kernel(
    _conv_g_body,
    out_type=jax.ShapeDtypeStruct((NCORE, NP, HH), jnp.float32),
    mesh=_MESH,
    compiler_params=_SC_PARAMS,
    scratch_types=_RING_SCRATCH + [
        pltpu.VMEM_SHARED((ACC_ROWS, HH), jnp.float32),
    ] + _SEMS,
)

_conv_pool = pl.kernel(
    _conv_pool_body,
    out_type=(jax.ShapeDtypeStruct((NCORE, NG, HH), jnp.float32),
              jax.ShapeDtypeStruct((NG, HH), jnp.float32)),
    mesh=_MESH,
    compiler_params=_SC_PARAMS,
    scratch_types=_RING_SCRATCH + [
        pltpu.VMEM((CH,), jnp.float32),
        pltpu.VMEM((CH,), jnp.int32),
        pltpu.VMEM_SHARED((ACC_ROWS, HH), jnp.float32),
        pltpu.VMEM_SHARED((PG, HH), jnp.float32),
        pltpu.VMEM_SHARED((PG, HH), jnp.float32),
    ] + _SEMS,
)


def _b1_body(deg_ref, x_ref, w_ref, dinv_ref, hp_ref):
    deg = deg_ref[0, :, 0:1] + deg_ref[1, :, 0:1]
    dinv = lax.rsqrt(deg)
    dinv_ref[...] = dinv[:, 0]
    h = jnp.dot(x_ref[...] * dinv, w_ref[...],
                preferred_element_type=jnp.float32)
    hp_ref[0] = h[:, :HH]
    hp_ref[1] = h[:, HH:]


_b1_call = pl.pallas_call(
    _b1_body,
    grid=(NP // 512,),
    in_specs=[
        pl.BlockSpec((NCORE, 512, L), lambda i: (0, i, 0)),
        pl.BlockSpec((512, IN_PAD), lambda i: (i, 0)),
        pl.BlockSpec((IN_PAD, HID), lambda i: (0, 0)),
    ],
    out_specs=[
        pl.BlockSpec((512,), lambda i: (i,)),
        pl.BlockSpec((NCORE, 512, HH), lambda i: (0, i, 0)),
    ],
    out_shape=[
        jax.ShapeDtypeStruct((NP,), jnp.float32),
        jax.ShapeDtypeStruct((NCORE, NP, HH), jnp.float32),
    ],
)


def _mm_body(agg_ref, dinv_ref, b_ref, w_ref, o_ref):
    dv = dinv_ref[...][:, None]
    t0 = jnp.maximum(agg_ref[0] * dv + b_ref[0, :HH], 0.0) * dv
    t1 = jnp.maximum(agg_ref[1] * dv + b_ref[0, HH:], 0.0) * dv
    h = (jnp.dot(t0, w_ref[:HH, :], preferred_element_type=jnp.float32)
         + jnp.dot(t1, w_ref[HH:, :], preferred_element_type=jnp.float32))
    o_ref[0] = h[:, :HH]
    o_ref[1] = h[:, HH:]


_mm_call = pl.pallas_call(
    _mm_body,
    grid=(NP // 512,),
    in_specs=[
        pl.BlockSpec((NCORE, 512, HH), lambda i: (0, i, 0)),
        pl.BlockSpec((512,), lambda i: (i,)),
        pl.BlockSpec((1, HID), lambda i: (0, 0)),
        pl.BlockSpec((HID, HID), lambda i: (0, 0)),
    ],
    out_specs=pl.BlockSpec((NCORE, 512, HH), lambda i: (0, i, 0)),
    out_shape=jax.ShapeDtypeStruct((NCORE, NP, HH), jnp.float32),
)


def _final_body(pp_ref, cp_ref, b3_ref, wl_ref, bl_ref, o_ref):
    cnt = jnp.maximum(cp_ref[:, 0:1], 1.0)
    p0 = pp_ref[0] / cnt + b3_ref[0, :HH]
    p1 = pp_ref[1] / cnt + b3_ref[0, HH:]
    o_ref[...] = (jnp.dot(p0, wl_ref[:HH, :],
                          preferred_element_type=jnp.float32)
                  + jnp.dot(p1, wl_ref[HH:, :],
                            preferred_element_type=jnp.float32)
                  + bl_ref[...])


_final_call = pl.pallas_call(
    _final_body,
    out_shape=jax.ShapeDtypeStruct((NG, 128), jnp.float32),
)


def kernel(x, edge_index, batch, W1, b1, W2, b2, W3, b3, Wl, bl):
    f32 = jnp.float32
    xpad = jnp.zeros((NP, IN_PAD), f32).at[:NN, :x.shape[1]].set(x)
    srcp = jnp.concatenate(
        [edge_index[0], jnp.zeros((EP - NE,), jnp.int32)]).reshape(ER, B)
    dstp = jnp.concatenate(
        [edge_index[1], jnp.full((EP - NE,), NP, jnp.int32)]).reshape(ER, B)
    batchp = jnp.concatenate(
        [batch, jnp.full((NP - NN,), NG, jnp.int32)])
    W1p = jnp.zeros((IN_PAD, HID), f32).at[:W1.shape[0]].set(W1)
    Wlp = jnp.zeros((HID, 128), f32).at[:, :Wl.shape[1]].set(Wl)
    blp = jnp.zeros((1, 128), f32).at[0, :bl.shape[0]].set(bl)

    deg = _deg_kernel(dstp)
    dinv, h1 = _b1_call(deg, xpad, W1p)
    agg1 = _conv_g(h1, srcp, dstp)
    h2 = _mm_call(agg1, dinv, b1.reshape(1, HID), W2)
    agg2 = _conv_g(h2, srcp, dstp)
    h3 = _mm_call(agg2, dinv, b2.reshape(1, HID), W3)
    pooledp, cntp = _conv_pool(h3, srcp, dstp, dinv, batchp)
    out = _final_call(pooledp, cntp, b3.reshape(1, HID), Wlp, blp)
    return out[:, :bl.shape[0]]


# speedup vs baseline: 1.0330x; 1.0330x over previous
import functools

import jax
import jax.numpy as jnp
from jax import lax
from jax.experimental import pallas as pl
from jax.experimental.pallas import tpu as pltpu
from jax.experimental.pallas import tpu_sc as plsc

NN = 50000
NE = 800000
NG = 512
HID = 64
HH = HID // 2
IN_PAD = 8

NCORE = 2
NSUB = 16
L = 16

NP = 50176
TRN = NP // NSUB
ACC_ROWS = NP + 8

EP = 819200
B = 128
ER = EP // B
CPB = 40
EITERS = ER // NSUB
NBLK = EITERS // CPB
DITERS = ER // NSUB // 2
DBLK = DITERS // CPB
NBUF = 4

CH = 112
NCH = TRN // CH

PG = 640
PTT = PG // NSUB

_MESH = plsc.VectorSubcoreMesh(
    core_axis_name="c", subcore_axis_name="s",
    num_cores=NCORE, num_subcores=NSUB)
_SC_PARAMS = pltpu.CompilerParams(use_tc_tiling_on_sc=False)


@functools.partial(
    pl.kernel,
    out_type=jax.ShapeDtypeStruct((NCORE, NP, L), jnp.float32),
    mesh=_MESH,
    compiler_params=_SC_PARAMS,
    scratch_types=[
        pltpu.VMEM((CPB, B), jnp.int32),
        pltpu.VMEM((B, L), jnp.float32),
        pltpu.VMEM((CH, L), jnp.float32),
        pltpu.VMEM_SHARED((ACC_ROWS, L), jnp.float32),
    ],
)
def _deg_kernel(dst_hbm, deg_hbm, dblk, ones, ibuf, cnt):
    c = lax.axis_index("c")
    s = lax.axis_index("s")

    def fill1(r, _):
        ones[r, :] = jnp.full((L,), 1.0, jnp.float32)
        return 0
    lax.fori_loop(0, B, fill1, 0)

    init = jnp.where(c == 0, 1.0, 0.0)

    def filli(r, _):
        ibuf[r, :] = jnp.full((L,), 1.0, jnp.float32) * init
        return 0
    lax.fori_loop(0, CH, filli, 0)

    def initl(k, _):
        pltpu.sync_copy(ibuf, cnt.at[pl.ds(s * TRN + k * CH, CH)])
        return 0
    lax.fori_loop(0, NCH, initl, 0)
    plsc.subcore_barrier()

    def blkloop(blk, _):
        row0 = c * (ER // 2) + s * DITERS + blk * CPB
        pltpu.sync_copy(dst_hbm.at[pl.ds(row0, CPB)], dblk)

        def cloop(jj, _):
            pltpu.sync_copy(ones, cnt.at[dblk.at[jj]], add=True)
            return 0
        lax.fori_loop(0, CPB, cloop, 0)
        return 0
    lax.fori_loop(0, DBLK, blkloop, 0)
    plsc.subcore_barrier()

    pltpu.sync_copy(cnt.at[pl.ds(s * TRN, TRN)],
                    deg_hbm.at[c, pl.ds(s * TRN, TRN)])


def _edge_ring(c, s, hp_hbm, src_hbm, dst_hbm, sblk, dblk, rows, gsem, tsem,
               acc):
    def blkloop(blk, _):
        row0 = s * EITERS + blk * CPB
        pltpu.sync_copy(src_hbm.at[pl.ds(row0, CPB)], sblk)
        pltpu.sync_copy(dst_hbm.at[pl.ds(row0, CPB)], dblk)
        pltpu.async_copy(hp_hbm.at[c].at[sblk.at[0]], rows[0], gsem[0])
        pltpu.async_copy(hp_hbm.at[c].at[sblk.at[1]], rows[1], gsem[1])

        def qloop(t, _):
            for u in range(NBUF):
                j = NBUF * t + u
                v = (u + 2) % NBUF

                @pl.when(j >= 2)
                def _():
                    pltpu.make_async_copy(
                        rows[v], acc.at[pl.ds(0, B)], tsem[v]).wait()

                @pl.when(j + 2 < CPB)
                def _():
                    pltpu.async_copy(
                        hp_hbm.at[c].at[sblk.at[j + 2]], rows[v], gsem[v])

                pltpu.make_async_copy(
                    hp_hbm.at[c, pl.ds(0, B)], rows[u], gsem[u]).wait()
                pltpu.async_copy(
                    rows[u], acc.at[dblk.at[j]], tsem[u], add=True)
            return 0
        lax.fori_loop(0, CPB // NBUF, qloop, 0)
        pltpu.make_async_copy(rows[2], acc.at[pl.ds(0, B)], tsem[2]).wait()
        pltpu.make_async_copy(rows[3], acc.at[pl.ds(0, B)], tsem[3]).wait()
        return 0
    lax.fori_loop(0, NBLK, blkloop, 0)


def _conv_g_body(hp_hbm, src_hbm, dst_hbm, g_hbm,
                 sblk, dblk, r0, r1, r2, r3, acc,
                 g0, g1, g2, g3, t0, t1, t2, t3):
    c = lax.axis_index("c")
    s = lax.axis_index("s")
    pltpu.sync_copy(hp_hbm.at[c, pl.ds(s * TRN, TRN)],
                    acc.at[pl.ds(s * TRN, TRN)])
    plsc.subcore_barrier()
    _edge_ring(c, s, hp_hbm, src_hbm, dst_hbm, sblk, dblk,
               (r0, r1, r2, r3), (g0, g1, g2, g3), (t0, t1, t2, t3), acc)
    plsc.subcore_barrier()
    pltpu.sync_copy(acc.at[pl.ds(s * TRN, TRN)],
                    g_hbm.at[c, pl.ds(s * TRN, TRN)])


def _conv_pool_body(hp_hbm, src_hbm, dst_hbm, dinv_hbm, batch_hbm,
                    pooled_hbm, cntg_hbm,
                    sblk, dblk, r0, r1, r2, r3, dchunk, bidx,
                    acc, pooled, cntg,
                    g0, g1, g2, g3, t0, t1, t2, t3):
    c = lax.axis_index("c")
    s = lax.axis_index("s")
    pltpu.sync_copy(hp_hbm.at[c, pl.ds(s * TRN, TRN)],
                    acc.at[pl.ds(s * TRN, TRN)])
    plsc.subcore_barrier()
    _edge_ring(c, s, hp_hbm, src_hbm, dst_hbm, sblk, dblk,
               (r0, r1, r2, r3), (g0, g1, g2, g3), (t0, t1, t2, t3), acc)

    def fillz(r, _):
        for j in range(HH // L):
            r3[r, pl.ds(j * L, L)] = jnp.zeros((L,), jnp.float32)
        return 0
    lax.fori_loop(0, PTT, fillz, 0)
    pltpu.sync_copy(r3.at[pl.ds(0, PTT)], pooled.at[pl.ds(s * PTT, PTT)])
    pltpu.sync_copy(r3.at[pl.ds(0, PTT)], cntg.at[pl.ds(s * PTT, PTT)])

    def fillo(r, _):
        for j in range(HH // L):
            r2[r, pl.ds(j * L, L)] = jnp.full((L,), 1.0, jnp.float32)
        return 0
    lax.fori_loop(0, CH, fillo, 0)
    plsc.subcore_barrier()

    def nloop(k, _):
        grow = s * TRN + k * CH
        pltpu.sync_copy(acc.at[pl.ds(grow, CH)], r0.at[pl.ds(0, CH)])
        pltpu.sync_copy(dinv_hbm.at[pl.ds(grow, CH)], dchunk)
        pltpu.sync_copy(batch_hbm.at[pl.ds(grow, CH)], bidx)

        def rblk(t, _):
            dv16 = dchunk[pl.ds(t * L, L)]
            for r16 in range(L):
                row = t * L + r16
                dv = dv16[r16]
                for j in range(HH // L):
                    sl = pl.ds(j * L, L)
                    r0[row, sl] = r0[row, sl] * dv
            return 0
        lax.fori_loop(0, CH // L, rblk, 0)

        pltpu.sync_copy(r0.at[pl.ds(0, CH)], pooled.at[bidx], add=True)

        @pl.when(c == 0)
        def _():
            pltpu.sync_copy(r2.at[pl.ds(0, CH)], cntg.at[bidx], add=True)
        return 0
    lax.fori_loop(0, NCH, nloop, 0)
    plsc.subcore_barrier()

    @pl.when(s == 0)
    def _():
        pltpu.sync_copy(pooled.at[pl.ds(0, NG)], pooled_hbm.at[c])

        @pl.when(c == 0)
        def _():
            pltpu.sync_copy(cntg.at[pl.ds(0, NG)], cntg_hbm)


_RING_SCRATCH = [
    pltpu.VMEM((CPB, B), jnp.int32),
    pltpu.VMEM((CPB, B), jnp.int32),
    pltpu.VMEM((B, HH), jnp.float32),
    pltpu.VMEM((B, HH), jnp.float32),
    pltpu.VMEM((B, HH), jnp.float32),
    pltpu.VMEM((B, HH), jnp.float32),
]
_SEMS = [pltpu.SemaphoreType.DMA] * (2 * NBUF)

_conv_g = pl.kernel(
    _conv_g_body,
    out_type=jax.ShapeDtypeStruct((NCORE, NP, HH), jnp.float32),
    mesh=_MESH,
    compiler_params=_SC_PARAMS,
    scratch_types=_RING_SCRATCH + [
        pltpu.VMEM_SHARED((ACC_ROWS, HH), jnp.float32),
    ] + _SEMS,
)

_conv_pool = pl.kernel(
    _conv_pool_body,
    out_type=(jax.ShapeDtypeStruct((NCORE, NG, HH), jnp.float32),
              jax.ShapeDtypeStruct((NG, HH), jnp.float32)),
    mesh=_MESH,
    compiler_params=_SC_PARAMS,
    scratch_types=_RING_SCRATCH + [
        pltpu.VMEM((CH,), jnp.float32),
        pltpu.VMEM((CH,), jnp.int32),
        pltpu.VMEM_SHARED((ACC_ROWS, HH), jnp.float32),
        pltpu.VMEM_SHARED((PG, HH), jnp.float32),
        pltpu.VMEM_SHARED((PG, HH), jnp.float32),
    ] + _SEMS,
)


def _b1_body(deg_ref, x_ref, w_ref, dinv_ref, hp_ref):
    deg = deg_ref[0, :, 0:1] + deg_ref[1, :, 0:1]
    dinv = lax.rsqrt(deg)
    dinv_ref[...] = dinv[:, 0]
    h = jnp.dot(x_ref[...] * dinv, w_ref[...],
                preferred_element_type=jnp.float32)
    hp_ref[0] = h[:, :HH]
    hp_ref[1] = h[:, HH:]


_b1_call = pl.pallas_call(
    _b1_body,
    grid=(NP // 512,),
    in_specs=[
        pl.BlockSpec((NCORE, 512, L), lambda i: (0, i, 0)),
        pl.BlockSpec((512, IN_PAD), lambda i: (i, 0)),
        pl.BlockSpec((IN_PAD, HID), lambda i: (0, 0)),
    ],
    out_specs=[
        pl.BlockSpec((512,), lambda i: (i,)),
        pl.BlockSpec((NCORE, 512, HH), lambda i: (0, i, 0)),
    ],
    out_shape=[
        jax.ShapeDtypeStruct((NP,), jnp.float32),
        jax.ShapeDtypeStruct((NCORE, NP, HH), jnp.float32),
    ],
)


def _mm_body(agg_ref, dinv_ref, b_ref, w_ref, o_ref):
    dv = dinv_ref[...][:, None]
    t0 = jnp.maximum(agg_ref[0] * dv + b_ref[0, :HH], 0.0) * dv
    t1 = jnp.maximum(agg_ref[1] * dv + b_ref[0, HH:], 0.0) * dv
    h = (jnp.dot(t0, w_ref[:HH, :], preferred_element_type=jnp.float32)
         + jnp.dot(t1, w_ref[HH:, :], preferred_element_type=jnp.float32))
    o_ref[0] = h[:, :HH]
    o_ref[1] = h[:, HH:]


_mm_call = pl.pallas_call(
    _mm_body,
    grid=(NP // 512,),
    in_specs=[
        pl.BlockSpec((NCORE, 512, HH), lambda i: (0, i, 0)),
        pl.BlockSpec((512,), lambda i: (i,)),
        pl.BlockSpec((1, HID), lambda i: (0, 0)),
        pl.BlockSpec((HID, HID), lambda i: (0, 0)),
    ],
    out_specs=pl.BlockSpec((NCORE, 512, HH), lambda i: (0, i, 0)),
    out_shape=jax.ShapeDtypeStruct((NCORE, NP, HH), jnp.float32),
)


def _final_body(pp_ref, cp_ref, b3_ref, wl_ref, bl_ref, o_ref):
    cnt = jnp.maximum(cp_ref[:, 0:1], 1.0)
    p0 = pp_ref[0] / cnt + b3_ref[0, :HH]
    p1 = pp_ref[1] / cnt + b3_ref[0, HH:]
    o_ref[...] = (jnp.dot(p0, wl_ref[:HH, :],
                          preferred_element_type=jnp.float32)
                  + jnp.dot(p1, wl_ref[HH:, :],
                            preferred_element_type=jnp.float32)
                  + bl_ref[...])


_final_call = pl.pallas_call(
    _final_body,
    out_shape=jax.ShapeDtypeStruct((NG, 128), jnp.float32),
)


def kernel(x, edge_index, batch, W1, b1, W2, b2, W3, b3, Wl, bl):
    f32 = jnp.float32
    xpad = jnp.zeros((NP, IN_PAD), f32).at[:NN, :x.shape[1]].set(x)
    srcp = jnp.concatenate(
        [edge_index[0], jnp.zeros((EP - NE,), jnp.int32)]).reshape(ER, B)
    dstp = jnp.concatenate(
        [edge_index[1], jnp.full((EP - NE,), NP, jnp.int32)]).reshape(ER, B)
    batchp = jnp.concatenate(
        [batch, jnp.full((NP - NN,), NG, jnp.int32)])
    W1p = jnp.zeros((IN_PAD, HID), f32).at[:W1.shape[0]].set(W1)
    Wlp = jnp.zeros((HID, 128), f32).at[:, :Wl.shape[1]].set(Wl)
    blp = jnp.zeros((1, 128), f32).at[0, :bl.shape[0]].set(bl)

    deg = _deg_kernel(dstp)
    dinv, h1 = _b1_call(deg, xpad, W1p)
    agg1 = _conv_g(h1, srcp, dstp)
    h2 = _mm_call(agg1, dinv, b1.reshape(1, HID), W2)
    agg2 = _conv_g(h2, srcp, dstp)
    h3 = _mm_call(agg2, dinv, b2.reshape(1, HID), W3)
    pooledp, cntp = _conv_pool(h3, srcp, dstp, dinv, batchp)
    out = _final_call(pooledp, cntp, b3.reshape(1, HID), Wlp, blp)
    return out[:, :bl.shape[0]]
